# baseline (speedup 1.0000x reference)
import jax
import jax.numpy as jnp
from jax.experimental import pallas as pl

N_NEI = 50
DELTA = 0.001
BB = 128
K = 10000
D = 128
A = 8
B = 1024


def _tree_sum(x):
    s1 = jnp.sum(x.reshape(10, K // 10, BB), axis=0)
    s2 = jnp.sum(s1.reshape(5, K // 50, BB), axis=0)
    return jnp.sum(s2, axis=0, keepdims=True)


def _body(qt_ref, keys_ref, vals_ref, q2_ref, k2_ref, max_ref, act_ref):
    a = pl.program_id(1)
    qt = qt_ref[...]
    keys = keys_ref[0]
    v = vals_ref[0]
    q2 = q2_ref[...]
    k2 = k2_ref[0]

    mm = jnp.dot(keys, qt, preferred_element_type=jnp.float32)
    d = (q2 + k2) - 2.0 * mm

    s = jax.lax.bitcast_convert_type(d, jnp.int32)
    ikey = s ^ ((s >> 31) & jnp.int32(0x7FFFFFFF))

    lo = jnp.min(ikey, axis=0, keepdims=True)
    hi = jnp.max(ikey, axis=0, keepdims=True)

    def step(_, carry):
        lo, hi = carry
        mid = (lo >> 1) + (hi >> 1) + (lo & hi & 1)
        cnt = _tree_sum((ikey <= mid).astype(jnp.int32))
        ge = cnt >= N_NEI
        return jnp.where(ge, lo, mid + 1), jnp.where(ge, mid, hi)

    carry = (lo, hi)
    for _ in range(32):
        carry = step(0, carry)
    lo, _ = carry

    idx = jax.lax.broadcasted_iota(jnp.int32, (K, 1), 0)
    selidx = jnp.where(ikey < lo, jnp.int32(-1),
                       jnp.where(ikey == lo, idx, jnp.int32(K)))
    cnt_le = _tree_sum((selidx < K).astype(jnp.int32))

    def istep(_, carry):
        lo2, hi2 = carry
        mid = (lo2 + hi2) >> 1
        cnt = _tree_sum((selidx <= mid).astype(jnp.int32))
        ge = cnt >= N_NEI
        return jnp.where(ge, lo2, mid + 1), jnp.where(ge, mid, hi2)

    def find_jstar(_):
        j, _ = jax.lax.fori_loop(0, 14, istep,
                                 (jnp.zeros_like(lo), jnp.full_like(lo, K - 1)))
        return j

    jstar = jax.lax.cond(jnp.any(cnt_le > N_NEI), find_jstar,
                         lambda _: jnp.full_like(lo, K - 1), None)

    mask = selidx <= jstar
    w = jnp.where(mask, pl.reciprocal(d + DELTA, approx=True), 0.0)
    wtot = _tree_sum(w)
    wval = jnp.dot(v, w, preferred_element_type=jnp.float32,
                   precision=jax.lax.Precision.HIGHEST)
    val = wval / wtot

    @pl.when(a == 0)
    def _():
        max_ref[0] = val
        act_ref[0] = jnp.zeros((1, BB), jnp.int32)

    @pl.when(a > 0)
    def _():
        better = val > max_ref[0]
        act_ref[0] = jnp.where(better, a, act_ref[0])
        max_ref[0] = jnp.where(better, val, max_ref[0])


def kernel(key, dnd_keys, dnd_values):
    qt = key.T
    q2 = jnp.sum(key * key, axis=1)[None, :]
    k2 = jnp.sum(dnd_keys * dnd_keys, axis=2)[:, :, None]
    nb = B // BB
    max_o, act_o = pl.pallas_call(
        _body,
        grid=(nb, A),
        in_specs=[
            pl.BlockSpec((D, BB), lambda i, a: (0, i)),
            pl.BlockSpec((1, K, D), lambda i, a: (a, 0, 0)),
            pl.BlockSpec((1, 1, K), lambda i, a: (a, 0, 0)),
            pl.BlockSpec((1, BB), lambda i, a: (0, i)),
            pl.BlockSpec((1, K, 1), lambda i, a: (a, 0, 0)),
        ],
        out_specs=[
            pl.BlockSpec((1, 1, BB), lambda i, a: (i, 0, 0)),
            pl.BlockSpec((1, 1, BB), lambda i, a: (i, 0, 0)),
        ],
        out_shape=[
            jax.ShapeDtypeStruct((nb, 1, BB), jnp.float32),
            jax.ShapeDtypeStruct((nb, 1, BB), jnp.int32),
        ],
    )(qt, dnd_keys, dnd_values.reshape(A, 1, K), q2, k2)
    return max_o.reshape(B, 1), act_o.reshape(B, 1)


# speedup vs baseline: 1.2109x; 1.2109x over previous
import jax
import jax.numpy as jnp
from jax.experimental import pallas as pl

N_NEI = 50
DELTA = 0.001
BB = 128
K = 10000
D = 128
A = 8
B = 1024


def _tree_sum(x):
    s1 = jnp.sum(x.reshape(10, K // 10, BB), axis=0)
    s2 = jnp.sum(s1.reshape(5, K // 50, BB), axis=0)
    return jnp.sum(s2, axis=0, keepdims=True)


def _body(qt_ref, keys_ref, vals_ref, q2_ref, k2_ref, max_ref, act_ref):
    a = pl.program_id(1)
    qt = qt_ref[...]
    keys = keys_ref[0]
    v = vals_ref[0]
    q2 = q2_ref[...]
    k2 = k2_ref[0]

    mm = jnp.dot(keys, qt, preferred_element_type=jnp.float32)
    d = (q2 + k2) - 2.0 * mm

    s = jax.lax.bitcast_convert_type(d, jnp.int32)
    ikey = s ^ ((s >> 31) & jnp.int32(0x7FFFFFFF))

    lo = jnp.min(ikey, axis=0, keepdims=True)
    hi = jnp.max(ikey, axis=0, keepdims=True)

    def step(_, carry):
        lo, hi = carry
        mid = (lo >> 1) + (hi >> 1) + (lo & hi & 1)
        cnt = _tree_sum((ikey <= mid).astype(jnp.int32))
        ge = cnt >= N_NEI
        return jnp.where(ge, lo, mid + 1), jnp.where(ge, mid, hi)

    lo, _ = jax.lax.fori_loop(0, 32, step, (lo, hi))

    idx = jax.lax.broadcasted_iota(jnp.int32, (K, 1), 0)
    selidx = jnp.where(ikey < lo, jnp.int32(-1),
                       jnp.where(ikey == lo, idx, jnp.int32(K)))
    cnt_le = _tree_sum((selidx < K).astype(jnp.int32))

    def istep(_, carry):
        lo2, hi2 = carry
        mid = (lo2 + hi2) >> 1
        cnt = _tree_sum((selidx <= mid).astype(jnp.int32))
        ge = cnt >= N_NEI
        return jnp.where(ge, lo2, mid + 1), jnp.where(ge, mid, hi2)

    def find_jstar(_):
        j, _ = jax.lax.fori_loop(0, 14, istep,
                                 (jnp.zeros_like(lo), jnp.full_like(lo, K - 1)))
        return j

    jstar = jax.lax.cond(jnp.any(cnt_le > N_NEI), find_jstar,
                         lambda _: jnp.full_like(lo, K - 1), None)

    mask = selidx <= jstar
    w = jnp.where(mask, pl.reciprocal(d + DELTA, approx=True), 0.0)
    wtot = _tree_sum(w)
    wval = jnp.dot(v, w, preferred_element_type=jnp.float32,
                   precision=jax.lax.Precision.HIGHEST)
    val = wval / wtot

    @pl.when(a == 0)
    def _():
        max_ref[0] = val
        act_ref[0] = jnp.zeros((1, BB), jnp.int32)

    @pl.when(a > 0)
    def _():
        better = val > max_ref[0]
        act_ref[0] = jnp.where(better, a, act_ref[0])
        max_ref[0] = jnp.where(better, val, max_ref[0])


def kernel(key, dnd_keys, dnd_values):
    qt = key.T
    q2 = jnp.sum(key * key, axis=1)[None, :]
    k2 = jnp.sum(dnd_keys * dnd_keys, axis=2)[:, :, None]
    nb = B // BB
    max_o, act_o = pl.pallas_call(
        _body,
        grid=(nb, A),
        in_specs=[
            pl.BlockSpec((D, BB), lambda i, a: (0, i)),
            pl.BlockSpec((1, K, D), lambda i, a: (a, 0, 0)),
            pl.BlockSpec((1, 1, K), lambda i, a: (a, 0, 0)),
            pl.BlockSpec((1, BB), lambda i, a: (0, i)),
            pl.BlockSpec((1, K, 1), lambda i, a: (a, 0, 0)),
        ],
        out_specs=[
            pl.BlockSpec((1, 1, BB), lambda i, a: (i, 0, 0)),
            pl.BlockSpec((1, 1, BB), lambda i, a: (i, 0, 0)),
        ],
        out_shape=[
            jax.ShapeDtypeStruct((nb, 1, BB), jnp.float32),
            jax.ShapeDtypeStruct((nb, 1, BB), jnp.int32),
        ],
    )(qt, dnd_keys, dnd_values.reshape(A, 1, K), q2, k2)
    return max_o.reshape(B, 1), act_o.reshape(B, 1)


# speedup vs baseline: 2.1061x; 1.7392x over previous
import jax
import jax.numpy as jnp
from jax.experimental import pallas as pl

N_NEI = 50
DELTA = 0.001
BB = 256
K = 10000
D = 128
A = 8
B = 1024


def _tree_sum(x):
    s1 = jnp.sum(x.reshape(10, K // 10, BB), axis=0)
    s2 = jnp.sum(s1.reshape(5, K // 50, BB), axis=0)
    return jnp.sum(s2, axis=0, keepdims=True)


def _body(qt_ref, keys_ref, vals_ref, q2_ref, k2_ref, max_ref, act_ref):
    a = pl.program_id(1)
    qt = qt_ref[...]
    keys = keys_ref[0]
    v = vals_ref[0]
    q2 = q2_ref[...]
    k2 = k2_ref[0]

    mm = jnp.dot(keys, qt, preferred_element_type=jnp.float32)
    d = (q2 + k2) - 2.0 * mm

    s = jax.lax.bitcast_convert_type(d, jnp.int32)
    ikey = s ^ ((s >> 31) & jnp.int32(0x7FFFFFFF))

    lo = jnp.min(ikey, axis=0, keepdims=True)
    hi = jnp.max(ikey, axis=0, keepdims=True)

    def step(_, carry):
        lo, hi = carry
        mid = (lo >> 1) + (hi >> 1) + (lo & hi & 1)
        cnt = _tree_sum((ikey <= mid).astype(jnp.int32))
        ge = cnt >= N_NEI
        return jnp.where(ge, lo, mid + 1), jnp.where(ge, mid, hi)

    lo, _ = jax.lax.fori_loop(0, 32, step, (lo, hi))

    idx = jax.lax.broadcasted_iota(jnp.int32, (K, 1), 0)
    selidx = jnp.where(ikey < lo, jnp.int32(-1),
                       jnp.where(ikey == lo, idx, jnp.int32(K)))
    cnt_le = _tree_sum((selidx < K).astype(jnp.int32))

    def istep(_, carry):
        lo2, hi2 = carry
        mid = (lo2 + hi2) >> 1
        cnt = _tree_sum((selidx <= mid).astype(jnp.int32))
        ge = cnt >= N_NEI
        return jnp.where(ge, lo2, mid + 1), jnp.where(ge, mid, hi2)

    def find_jstar(_):
        j, _ = jax.lax.fori_loop(0, 14, istep,
                                 (jnp.zeros_like(lo), jnp.full_like(lo, K - 1)))
        return j

    jstar = jax.lax.cond(jnp.any(cnt_le > N_NEI), find_jstar,
                         lambda _: jnp.full_like(lo, K - 1), None)

    mask = selidx <= jstar
    w = jnp.where(mask, pl.reciprocal(d + DELTA, approx=True), 0.0)
    wtot = _tree_sum(w)
    wval = jnp.dot(v, w, preferred_element_type=jnp.float32,
                   precision=jax.lax.Precision.HIGHEST)
    val = wval / wtot

    @pl.when(a == 0)
    def _():
        max_ref[0] = val
        act_ref[0] = jnp.zeros((1, BB), jnp.int32)

    @pl.when(a > 0)
    def _():
        better = val > max_ref[0]
        act_ref[0] = jnp.where(better, a, act_ref[0])
        max_ref[0] = jnp.where(better, val, max_ref[0])


def kernel(key, dnd_keys, dnd_values):
    qt = key.T
    q2 = jnp.sum(key * key, axis=1)[None, :]
    k2 = jnp.sum(dnd_keys * dnd_keys, axis=2)[:, :, None]
    nb = B // BB
    max_o, act_o = pl.pallas_call(
        _body,
        grid=(nb, A),
        in_specs=[
            pl.BlockSpec((D, BB), lambda i, a: (0, i)),
            pl.BlockSpec((1, K, D), lambda i, a: (a, 0, 0)),
            pl.BlockSpec((1, 1, K), lambda i, a: (a, 0, 0)),
            pl.BlockSpec((1, BB), lambda i, a: (0, i)),
            pl.BlockSpec((1, K, 1), lambda i, a: (a, 0, 0)),
        ],
        out_specs=[
            pl.BlockSpec((1, 1, BB), lambda i, a: (i, 0, 0)),
            pl.BlockSpec((1, 1, BB), lambda i, a: (i, 0, 0)),
        ],
        out_shape=[
            jax.ShapeDtypeStruct((nb, 1, BB), jnp.float32),
            jax.ShapeDtypeStruct((nb, 1, BB), jnp.int32),
        ],
    )(qt, dnd_keys, dnd_values.reshape(A, 1, K), q2, k2)
    return max_o.reshape(B, 1), act_o.reshape(B, 1)


# speedup vs baseline: 2.7866x; 1.3231x over previous
import jax
import jax.numpy as jnp
from jax.experimental import pallas as pl

N_NEI = 50
DELTA = 0.001
BB = 512
K = 10000
D = 128
A = 8
B = 1024


def _tree_sum(x):
    s1 = jnp.sum(x.reshape(10, K // 10, BB), axis=0)
    s2 = jnp.sum(s1.reshape(5, K // 50, BB), axis=0)
    return jnp.sum(s2, axis=0, keepdims=True)


def _body(qt_ref, keys_ref, vals_ref, q2_ref, max_ref, act_ref):
    a = pl.program_id(1)
    qt = qt_ref[...]
    keys = keys_ref[0]
    v = vals_ref[0]
    q2 = q2_ref[...]
    k2 = jnp.sum(keys * keys, axis=1, keepdims=True)

    mm = jnp.dot(keys, qt, preferred_element_type=jnp.float32)
    d = (q2 + k2) - 2.0 * mm

    s = jax.lax.bitcast_convert_type(d, jnp.int32)
    ikey = s ^ ((s >> 31) & jnp.int32(0x7FFFFFFF))

    lo = jnp.min(ikey, axis=0, keepdims=True)
    hi = jnp.max(ikey, axis=0, keepdims=True)

    def step(_, carry):
        lo, hi = carry
        mid = (lo >> 1) + (hi >> 1) + (lo & hi & 1)
        cnt = _tree_sum((ikey <= mid).astype(jnp.int32))
        ge = cnt >= N_NEI
        return jnp.where(ge, lo, mid + 1), jnp.where(ge, mid, hi)

    lo, _ = jax.lax.fori_loop(0, 32, step, (lo, hi))

    idx = jax.lax.broadcasted_iota(jnp.int32, (K, 1), 0)
    cnt_le = _tree_sum((ikey <= lo).astype(jnp.int32))

    def istep(_, carry):
        lo2, hi2 = carry
        mid = (lo2 + hi2) >> 1
        cnt = _tree_sum(((ikey < lo) | ((ikey == lo) & (idx <= mid)))
                        .astype(jnp.int32))
        ge = cnt >= N_NEI
        return jnp.where(ge, lo2, mid + 1), jnp.where(ge, mid, hi2)

    def find_jstar(_):
        j, _ = jax.lax.fori_loop(0, 14, istep,
                                 (jnp.zeros_like(lo), jnp.full_like(lo, K - 1)))
        return j

    jstar = jax.lax.cond(jnp.any(cnt_le > N_NEI), find_jstar,
                         lambda _: jnp.full_like(lo, K - 1), None)

    mask = (ikey < lo) | ((ikey == lo) & (idx <= jstar))
    w = jnp.where(mask, pl.reciprocal(d + DELTA, approx=True), 0.0)
    wtot = _tree_sum(w)
    wval = jnp.dot(v, w, preferred_element_type=jnp.float32,
                   precision=jax.lax.Precision.HIGHEST)
    val = wval / wtot

    @pl.when(a == 0)
    def _():
        max_ref[0] = val
        act_ref[0] = jnp.zeros((1, BB), jnp.int32)

    @pl.when(a > 0)
    def _():
        better = val > max_ref[0]
        act_ref[0] = jnp.where(better, a, act_ref[0])
        max_ref[0] = jnp.where(better, val, max_ref[0])


def kernel(key, dnd_keys, dnd_values):
    qt = key.T
    q2 = jnp.sum(key * key, axis=1)[None, :]
    nb = B // BB
    max_o, act_o = pl.pallas_call(
        _body,
        grid=(nb, A),
        in_specs=[
            pl.BlockSpec((D, BB), lambda i, a: (0, i)),
            pl.BlockSpec((1, K, D), lambda i, a: (a, 0, 0)),
            pl.BlockSpec((1, 1, K), lambda i, a: (a, 0, 0)),
            pl.BlockSpec((1, BB), lambda i, a: (0, i)),
        ],
        out_specs=[
            pl.BlockSpec((1, 1, BB), lambda i, a: (i, 0, 0)),
            pl.BlockSpec((1, 1, BB), lambda i, a: (i, 0, 0)),
        ],
        out_shape=[
            jax.ShapeDtypeStruct((nb, 1, BB), jnp.float32),
            jax.ShapeDtypeStruct((nb, 1, BB), jnp.int32),
        ],
    )(qt, dnd_keys, dnd_values.reshape(A, 1, K), q2)
    return max_o.reshape(B, 1), act_o.reshape(B, 1)
